# baseline (speedup 1.0000x reference)
import functools

import jax
import jax.numpy as jnp
from jax import lax
from jax.experimental import pallas as pl
from jax.experimental.pallas import tpu as pltpu
from jax.experimental.pallas import tpu_sc as plsc

D = 128
CH = 128
NB = 5
NC = 2
NS = 16
NW = NC * NS


@functools.lru_cache(maxsize=None)
def _make_gather(n_total: int):
  n_per_w = n_total // NW
  n_gathers = n_per_w // CH
  assert n_gathers % NB == 0

  def body(ids_hbm, table_hbm, out_hbm, idx_v, rows_v, gsems, wsems):
    wid = lax.axis_index("s") * NC + lax.axis_index("c")
    base = wid * n_per_w
    pltpu.sync_copy(ids_hbm.at[pl.ds(base, n_per_w)], idx_v)

    def gather(g, s):
      return pltpu.make_async_copy(
          table_hbm.at[idx_v.at[pl.ds(g * CH, CH)]], rows_v.at[s],
          gsems[s])

    def write(g, s):
      return pltpu.make_async_copy(
          rows_v.at[s], out_hbm.at[pl.ds(base + g * CH, CH)], wsems[s])

    for b in range(NB):
      gather(b, b).start()
    gather(0, 0).wait()
    write(0, 0).start()

    @pl.loop(1, n_gathers - NB + 1, step=NB)
    def _(gb):
      for j in range(NB):
        g = gb + j
        s = (1 + j) % NB
        sh = (s + NB - 1) % NB
        write(g - 1, sh).wait()
        gather(g + NB - 1, sh).start()
        gather(g, s).wait()
        write(g, s).start()

    for g in range(n_gathers - NB + 1, n_gathers):
      s = g % NB
      gather(g, s).wait()
      write(g, s).start()
    for g in range(n_gathers - NB, n_gathers):
      write(g, g % NB).wait()

  return pl.kernel(
      body,
      out_type=jax.ShapeDtypeStruct((n_total, D), jnp.float32),
      mesh=plsc.VectorSubcoreMesh(core_axis_name="c", subcore_axis_name="s"),
      scratch_types=[
          pltpu.VMEM((n_per_w,), jnp.int32),
          pltpu.VMEM((NB, CH, D), jnp.float32),
          [pltpu.SemaphoreType.DMA] * NB,
          [pltpu.SemaphoreType.DMA] * NB,
      ],
  )


def _tc_copy_body(x_ref, o_ref):
  o_ref[...] = x_ref[...]


@functools.lru_cache(maxsize=None)
def _make_tc_copy(n_rows: int):
  blk = 8192
  return pl.pallas_call(
      _tc_copy_body,
      grid=(n_rows // blk,),
      in_specs=[pl.BlockSpec((blk, D), lambda i: (i, 0))],
      out_specs=pl.BlockSpec((blk, D), lambda i: (i, 0)),
      out_shape=jax.ShapeDtypeStruct((n_rows, D), jnp.float32),
  )


def kernel(input_ids, table):
  b, s = input_ids.shape
  ids = input_ids.reshape(-1).astype(jnp.int32)
  sc_out = _make_gather(b * s)(ids, table)
  tc_out = _make_tc_copy(98304)(table[:98304])
  patch = (tc_out[:1, :1] * 0.0).astype(jnp.float32)
  out = jax.lax.dynamic_update_slice(
      sc_out, sc_out[:1, :1] + patch, (0, 0))
  return out.reshape(b, s, D)


# speedup vs baseline: 1.1895x; 1.1895x over previous
import functools

import jax
import jax.numpy as jnp
from jax import lax
from jax.experimental import pallas as pl
from jax.experimental.pallas import tpu as pltpu
from jax.experimental.pallas import tpu_sc as plsc

D = 128
CH = 128
NB = 5
NC = 2
NS = 16
NW = NC * NS


@functools.lru_cache(maxsize=None)
def _make_gather(n_total: int):
  n_per_w = n_total // NW
  n_gathers = n_per_w // CH
  assert n_per_w % CH == 0 and n_gathers % NB == 0 and n_gathers >= 2 * NB

  def body(ids_hbm, table_hbm, out_hbm, idx_v, rows_v, gsems, wsems):
    wid = lax.axis_index("s") * NC + lax.axis_index("c")
    base = wid * n_per_w
    pltpu.sync_copy(ids_hbm.at[pl.ds(base, n_per_w)], idx_v)

    def gather(g, s):
      return pltpu.make_async_copy(
          table_hbm.at[idx_v.at[pl.ds(g * CH, CH)]], rows_v.at[s],
          gsems[s])

    def write(g, s):
      return pltpu.make_async_copy(
          rows_v.at[s], out_hbm.at[pl.ds(base + g * CH, CH)], wsems[s])

    for b in range(NB):
      gather(b, b).start()
    gather(0, 0).wait()
    write(0, 0).start()

    @pl.loop(1, n_gathers - NB + 1, step=NB)
    def _(gb):
      for j in range(NB):
        g = gb + j
        s = (1 + j) % NB
        sh = (s + NB - 1) % NB
        write(g - 1, sh).wait()
        gather(g + NB - 1, sh).start()
        gather(g, s).wait()
        write(g, s).start()

    for g in range(n_gathers - NB + 1, n_gathers):
      s = g % NB
      gather(g, s).wait()
      write(g, s).start()
    for g in range(n_gathers - NB, n_gathers):
      write(g, g % NB).wait()

  return pl.kernel(
      body,
      out_type=jax.ShapeDtypeStruct((n_total, D), jnp.float32),
      mesh=plsc.VectorSubcoreMesh(core_axis_name="c", subcore_axis_name="s"),
      scratch_types=[
          pltpu.VMEM((n_per_w,), jnp.int32),
          pltpu.VMEM((NB, CH, D), jnp.float32),
          [pltpu.SemaphoreType.DMA] * NB,
          [pltpu.SemaphoreType.DMA] * NB,
      ],
  )


def kernel(input_ids, table):
  b, s = input_ids.shape
  ids = input_ids.reshape(-1).astype(jnp.int32)
  out = _make_gather(b * s)(ids, table)
  return out.reshape(b, s, D)


# speedup vs baseline: 1.1920x; 1.0021x over previous
import functools

import jax
import jax.numpy as jnp
from jax import lax
from jax.experimental import pallas as pl
from jax.experimental.pallas import tpu as pltpu
from jax.experimental.pallas import tpu_sc as plsc

D = 128
CH = 128
NB = 5
NC = 2
NS = 16
NW = NC * NS


@functools.lru_cache(maxsize=None)
def _make_gather(n_total: int):
  n_per_w = n_total // NW
  n_gathers = n_per_w // CH
  assert n_per_w % CH == 0 and n_gathers % NB == 0 and n_gathers >= 2 * NB

  def body(ids_hbm, table_hbm, out_hbm, idx_v, rows_v, gsems, wsems):
    wid = lax.axis_index("s") * NC + lax.axis_index("c")
    base = wid * n_per_w
    pltpu.sync_copy(ids_hbm.at[pl.ds(base, n_per_w)], idx_v)

    def gather(g, s):
      return pltpu.make_async_copy(
          table_hbm.at[idx_v.at[pl.ds(g * CH, CH)]], rows_v.at[s],
          gsems[s])

    def write(g, s):
      return pltpu.make_async_copy(
          rows_v.at[s], out_hbm.at[pl.ds(base + g * CH, CH)], wsems[s])

    for b in range(NB):
      gather(b, b).start()
    gather(0, 0).wait()
    write(0, 0).start()

    @pl.loop(1, n_gathers - NB + 1, step=NB)
    def _(gb):
      for j in range(NB):
        g = gb + j
        s = (1 + j) % NB
        sh = (s + NB - 1) % NB
        gather(g, s).wait()
        write(g, s).start()
        write(g - 1, sh).wait()
        gather(g + NB - 1, sh).start()

    for g in range(n_gathers - NB + 1, n_gathers):
      s = g % NB
      gather(g, s).wait()
      write(g, s).start()
    for g in range(n_gathers - NB, n_gathers):
      write(g, g % NB).wait()

  return pl.kernel(
      body,
      out_type=jax.ShapeDtypeStruct((n_total, D), jnp.float32),
      mesh=plsc.VectorSubcoreMesh(core_axis_name="c", subcore_axis_name="s"),
      scratch_types=[
          pltpu.VMEM((n_per_w,), jnp.int32),
          pltpu.VMEM((NB, CH, D), jnp.float32),
          [pltpu.SemaphoreType.DMA] * NB,
          [pltpu.SemaphoreType.DMA] * NB,
      ],
  )


def kernel(input_ids, table):
  b, s = input_ids.shape
  ids = input_ids.reshape(-1).astype(jnp.int32)
  out = _make_gather(b * s)(ids, table)
  return out.reshape(b, s, D)
